# baseline (speedup 1.0000x reference)
import jax
import jax.numpy as jnp
from jax.experimental import pallas as pl
from jax.experimental.pallas import tpu as pltpu
from jax.experimental.pallas import tpu_sc as plsc

_CHUNK = 128
_SPLIT = 2
_SEQ_BLK = 8


def _sc_gather_flat(table_p, idx, n, dp):
    mesh = plsc.VectorSubcoreMesh(core_axis_name="c", subcore_axis_name="s")

    @pl.kernel(
        out_type=jax.ShapeDtypeStruct((n, dp), table_p.dtype),
        mesh=mesh,
    )
    def k(table_hbm, idx_hbm, out_hbm):
        def body(i_vmem, o_vmem):
            pltpu.sync_copy(table_hbm.at[i_vmem.at[0]], o_vmem)

        pltpu.emit_pipeline(
            body,
            grid=(n // _CHUNK,),
            in_specs=[pl.BlockSpec((1, _CHUNK), lambda i: (0, i))],
            out_specs=[pl.BlockSpec((_CHUNK, dp), lambda i: (i, 0))],
            core_axis_name=("c", "s"),
            dimension_semantics=(pltpu.PARALLEL,),
        )(idx_hbm, out_hbm)

    return k(table_p, idx)


def _tc_narrow(flat, bs, h, d):
    dp = flat.shape[1]
    rows = _SEQ_BLK * h

    def body(i_ref, o_ref):
        for j in range(_SEQ_BLK):
            o_ref[j] = i_ref[pl.ds(j * h, h), :d]

    return pl.pallas_call(
        body,
        out_shape=jax.ShapeDtypeStruct((bs, h, d), flat.dtype),
        grid=(bs // _SEQ_BLK,),
        in_specs=[pl.BlockSpec((rows, dp), lambda g: (g, 0))],
        out_specs=pl.BlockSpec((_SEQ_BLK, h, d), lambda g: (g, 0, 0)),
    )(flat)


def kernel(word_ids, table):
    B, H = word_ids.shape
    V, D = table.shape
    Dp = 384
    table_p = jnp.pad(table, ((0, 0), (0, Dp - D)))

    Bs = B // _SPLIT
    parts = []
    for s in range(_SPLIT):
        ids = word_ids[s * Bs:(s + 1) * Bs]
        idx = ids.reshape(1, Bs * H).astype(jnp.int32)
        flat = _sc_gather_flat(table_p, idx, Bs * H, Dp)
        parts.append(_tc_narrow(flat, Bs, H, D))
    return jnp.concatenate(parts, axis=0)


# speedup vs baseline: 1.1556x; 1.1556x over previous
import jax
import jax.numpy as jnp
from jax.experimental import pallas as pl
from jax.experimental.pallas import tpu as pltpu
from jax.experimental.pallas import tpu_sc as plsc

_CHUNK = 128
_SPLIT = 1
_SEQ_BLK = 8


def _sc_gather_flat(table_p, idx, n, dp):
    mesh = plsc.VectorSubcoreMesh(core_axis_name="c", subcore_axis_name="s")

    @pl.kernel(
        out_type=jax.ShapeDtypeStruct((n, dp), table_p.dtype),
        mesh=mesh,
    )
    def k(table_hbm, idx_hbm, out_hbm):
        def body(i_vmem, o_vmem):
            pltpu.sync_copy(table_hbm.at[i_vmem.at[0]], o_vmem)

        pltpu.emit_pipeline(
            body,
            grid=(n // _CHUNK,),
            in_specs=[pl.BlockSpec((1, _CHUNK), lambda i: (0, i))],
            out_specs=[pl.BlockSpec((_CHUNK, dp), lambda i: (i, 0))],
            core_axis_name=("c", "s"),
            dimension_semantics=(pltpu.PARALLEL,),
        )(idx_hbm, out_hbm)

    return k(table_p, idx)


def _tc_narrow(flat, bs, h, d):
    dp = flat.shape[1]
    rows = _SEQ_BLK * h

    def body(i_ref, o_ref):
        for j in range(_SEQ_BLK):
            o_ref[j] = i_ref[pl.ds(j * h, h), :d]

    return pl.pallas_call(
        body,
        out_shape=jax.ShapeDtypeStruct((bs, h, d), flat.dtype),
        grid=(bs // _SEQ_BLK,),
        in_specs=[pl.BlockSpec((rows, dp), lambda g: (g, 0))],
        out_specs=pl.BlockSpec((_SEQ_BLK, h, d), lambda g: (g, 0, 0)),
    )(flat)


def kernel(word_ids, table):
    B, H = word_ids.shape
    V, D = table.shape
    Dp = 384
    table_p = jnp.pad(table, ((0, 0), (0, Dp - D)))

    Bs = B // _SPLIT
    parts = []
    for s in range(_SPLIT):
        ids = word_ids[s * Bs:(s + 1) * Bs]
        idx = ids.reshape(1, Bs * H).astype(jnp.int32)
        flat = _sc_gather_flat(table_p, idx, Bs * H, Dp)
        parts.append(_tc_narrow(flat, Bs, H, D))
    if _SPLIT == 1:
        return parts[0]
    return jnp.concatenate(parts, axis=0)


# speedup vs baseline: 1.4151x; 1.2246x over previous
import jax
import jax.numpy as jnp
from jax.experimental import pallas as pl
from jax.experimental.pallas import tpu as pltpu
from jax.experimental.pallas import tpu_sc as plsc

_CHUNK = 128


def kernel(word_ids, table):
    B, H = word_ids.shape
    V, D = table.shape
    N = B * H
    Dp = 384
    idx = word_ids.reshape(1, N).astype(jnp.int32)
    table_p = jnp.pad(table, ((0, 0), (0, Dp - D)))
    mesh = plsc.VectorSubcoreMesh(core_axis_name="c", subcore_axis_name="s")

    @pl.kernel(
        out_type=jax.ShapeDtypeStruct((N, Dp), table.dtype),
        mesh=mesh,
    )
    def k(table_hbm, idx_hbm, out_hbm):
        def body(i_vmem, o_vmem):
            pltpu.sync_copy(table_hbm.at[i_vmem.at[0]], o_vmem)

        pltpu.emit_pipeline(
            body,
            grid=(N // _CHUNK,),
            in_specs=[pl.BlockSpec((1, _CHUNK), lambda i: (0, i))],
            out_specs=[pl.BlockSpec((_CHUNK, Dp), lambda i: (i, 0))],
            core_axis_name=("c", "s"),
            dimension_semantics=(pltpu.PARALLEL,),
        )(idx_hbm, out_hbm)

    out = k(table_p, idx)
    return out[:, :D].reshape(B, H, D)
